# baseline (speedup 1.0000x reference)
import functools

import jax
import jax.numpy as jnp
from jax import lax
from jax.experimental import pallas as pl
from jax.experimental.pallas import tpu as pltpu
from jax.experimental.pallas import tpu_sc as plsc

DIM = 4096
N_EXPERTS = 64
TOP_K = 8
TOKEN_BLOCK = 1024

N_WORKERS = 32
L = 16
CHUNK = 128


def _scores_kernel(w_ref, x_ref, bias_ref, s_out_ref, b_out_ref):
    logits_t = jax.lax.dot_general(
        w_ref[...], x_ref[...], (((1,), (1,)), ((), ())),
        preferred_element_type=jnp.float32,
    )
    sp = jnp.maximum(logits_t, 0.0) + jnp.log1p(jnp.exp(-jnp.abs(logits_t)))
    scores_t = jnp.sqrt(sp)
    s_out_ref[...] = scores_t.T
    b_out_ref[...] = (scores_t + bias_ref[...]).T


def _make_sc_topk(n_tokens):
    tok_per_w = n_tokens // N_WORKERS
    n_chunks = tok_per_w // CHUNK
    mesh = plsc.VectorSubcoreMesh(core_axis_name="c", subcore_axis_name="s")

    @functools.partial(
        pl.kernel,
        mesh=mesh,
        compiler_params=pltpu.CompilerParams(needs_layout_passes=False),
        out_type=[
            jax.ShapeDtypeStruct((TOP_K * n_tokens,), jnp.float32),
            jax.ShapeDtypeStruct((TOP_K * n_tokens,), jnp.int32),
        ],
        scratch_types=[
            pltpu.VMEM((CHUNK * N_EXPERTS,), jnp.float32),
            pltpu.VMEM((CHUNK * N_EXPERTS,), jnp.float32),
            pltpu.VMEM((TOP_K * CHUNK,), jnp.float32),
            pltpu.VMEM((TOP_K * CHUNK,), jnp.int32),
        ],
    )
    def sc_topk(scores_hbm, biased_hbm, w_out, idx_out, sbuf, bbuf,
                wout_v, iout_v):
        wid = lax.axis_index("s") * 2 + lax.axis_index("c")
        base = wid * tok_per_w
        lane = lax.iota(jnp.int32, L)
        neg_inf = jnp.full((L,), -jnp.inf, jnp.float32)
        zeros_i = jnp.zeros((L,), jnp.int32)
        ones_i = jnp.ones((L,), jnp.int32)

        def chunk_body(c, carry):
            row0 = base + c * CHUNK
            pltpu.sync_copy(
                scores_hbm.at[pl.ds(row0 * N_EXPERTS, CHUNK * N_EXPERTS)], sbuf
            )
            pltpu.sync_copy(
                biased_hbm.at[pl.ds(row0 * N_EXPERTS, CHUNK * N_EXPERTS)], bbuf
            )

            def group_body(g, inner):
                tok = g * L + lane
                tbase = tok * N_EXPERTS
                sel_lo = zeros_i
                sel_hi = zeros_i
                w_ks = []
                i_ks = []
                for k in range(TOP_K):
                    m = neg_inf
                    eidx = zeros_i
                    for e in range(N_EXPERTS):
                        e_vec = jnp.full((L,), e, jnp.int32)
                        b_e = plsc.load_gather(bbuf, [tbase + e])
                        sel_word = sel_lo if e < 32 else sel_hi
                        unused = (
                            lax.shift_right_logical(
                                sel_word, jnp.full((L,), e % 32, jnp.int32)
                            ) & ones_i
                        ) == zeros_i
                        take = (b_e > m) & unused
                        m = jnp.where(take, b_e, m)
                        eidx = jnp.where(take, e_vec, eidx)
                    w_k = plsc.load_gather(sbuf, [tbase + eidx])
                    bit_lo = jnp.where(
                        eidx < 32, lax.shift_left(ones_i, eidx), zeros_i
                    )
                    bit_hi = jnp.where(
                        eidx >= 32, lax.shift_left(ones_i, eidx - 32), zeros_i
                    )
                    sel_lo = sel_lo | bit_lo
                    sel_hi = sel_hi | bit_hi
                    w_ks.append(w_k)
                    i_ks.append(eidx)
                tot = w_ks[0]
                for k in range(1, TOP_K):
                    tot = tot + w_ks[k]
                for k in range(TOP_K):
                    plsc.store_scatter(wout_v, [k * CHUNK + tok], w_ks[k] / tot)
                    plsc.store_scatter(iout_v, [k * CHUNK + tok], i_ks[k])
                return inner

            lax.fori_loop(0, CHUNK // L, group_body, 0)
            for k in range(TOP_K):
                pltpu.sync_copy(
                    wout_v.at[pl.ds(k * CHUNK, CHUNK)],
                    w_out.at[pl.ds(k * n_tokens + row0, CHUNK)],
                )
                pltpu.sync_copy(
                    iout_v.at[pl.ds(k * CHUNK, CHUNK)],
                    idx_out.at[pl.ds(k * n_tokens + row0, CHUNK)],
                )
            return carry

        lax.fori_loop(0, n_chunks, chunk_body, 0)

    return sc_topk


@jax.jit
def kernel(x, weight, bias):
    n_tokens = x.shape[0]
    grid = (n_tokens // TOKEN_BLOCK,)
    bias2 = bias.reshape(N_EXPERTS, 1)
    scores, biased = pl.pallas_call(
        _scores_kernel,
        grid=grid,
        in_specs=[
            pl.BlockSpec((N_EXPERTS, DIM), lambda i: (0, 0)),
            pl.BlockSpec((TOKEN_BLOCK, DIM), lambda i: (i, 0)),
            pl.BlockSpec((N_EXPERTS, 1), lambda i: (0, 0)),
        ],
        out_specs=[
            pl.BlockSpec((TOKEN_BLOCK, N_EXPERTS), lambda i: (i, 0)),
            pl.BlockSpec((TOKEN_BLOCK, N_EXPERTS), lambda i: (i, 0)),
        ],
        out_shape=[
            jax.ShapeDtypeStruct((n_tokens, N_EXPERTS), jnp.float32),
            jax.ShapeDtypeStruct((n_tokens, N_EXPERTS), jnp.float32),
        ],
    )(weight, x, bias2)
    w1d, i1d = _make_sc_topk(n_tokens)(scores.reshape(-1), biased.reshape(-1))
    wsel = w1d.reshape(TOP_K, n_tokens).T
    idx = i1d.reshape(TOP_K, n_tokens).T
    return wsel, idx


# speedup vs baseline: 2.7440x; 2.7440x over previous
import jax
import jax.numpy as jnp
from jax.experimental import pallas as pl

DIM = 4096
N_EXPERTS = 64
TOP_K = 8
TOKEN_BLOCK = 1024


def _gate_kernel(w_ref, x_ref, bias_ref, w_out_ref, idx_out_ref):
    w = w_ref[...]
    x = x_ref[...]
    logits = jax.lax.dot_general(
        w, x, (((1,), (1,)), ((), ())), preferred_element_type=jnp.float32
    )
    sp = jnp.maximum(logits, 0.0) + jnp.log1p(jnp.exp(-jnp.abs(logits)))
    scores = jnp.sqrt(sp)
    biased = scores + bias_ref[...]

    row = jax.lax.broadcasted_iota(jnp.int32, biased.shape, 0)
    rev_row_f = jnp.float32(N_EXPERTS - 1) - row.astype(jnp.float32)
    cur = biased
    neg_inf = jnp.float32(-jnp.inf)
    w_rows = []
    i_rows = []
    for _ in range(TOP_K):
        m = jnp.max(cur, axis=0, keepdims=True)
        is_max = cur == m
        rev = jnp.max(jnp.where(is_max, rev_row_f, -1.0), axis=0, keepdims=True)
        onehot = rev_row_f == rev
        w_rows.append(jnp.sum(jnp.where(onehot, scores, 0.0), axis=0, keepdims=True))
        i_rows.append(jnp.float32(N_EXPERTS - 1) - rev)
        cur = jnp.where(onehot, neg_inf, cur)

    wsel = jnp.concatenate(w_rows, axis=0)
    idx = jnp.concatenate(i_rows, axis=0).astype(jnp.int32)
    wsel = wsel / jnp.sum(wsel, axis=0, keepdims=True)
    w_out_ref[...] = wsel
    idx_out_ref[...] = idx


@jax.jit
def kernel(x, weight, bias):
    n_tokens = x.shape[0]
    bias2 = bias.reshape(N_EXPERTS, 1)
    grid = (n_tokens // TOKEN_BLOCK,)
    wsel, idx = pl.pallas_call(
        _gate_kernel,
        grid=grid,
        in_specs=[
            pl.BlockSpec((N_EXPERTS, DIM), lambda i: (0, 0)),
            pl.BlockSpec((TOKEN_BLOCK, DIM), lambda i: (i, 0)),
            pl.BlockSpec((N_EXPERTS, 1), lambda i: (0, 0)),
        ],
        out_specs=[
            pl.BlockSpec((TOP_K, TOKEN_BLOCK), lambda i: (0, i)),
            pl.BlockSpec((TOP_K, TOKEN_BLOCK), lambda i: (0, i)),
        ],
        out_shape=[
            jax.ShapeDtypeStruct((TOP_K, n_tokens), jnp.float32),
            jax.ShapeDtypeStruct((TOP_K, n_tokens), jnp.int32),
        ],
    )(weight, x, bias2)
    return wsel.T, idx.T
